# baseline (speedup 1.0000x reference)
import functools

import jax
import jax.numpy as jnp
from jax import lax
from jax.experimental import pallas as pl
from jax.experimental.pallas import tpu as pltpu

_NUM_CLASSES = 21
_BG = 20
_RATIO = 3
_N = 131072
_R = 2048


def _tc_body(cats_ref, bbs_ref, gtb_ref, gt_ref, sel_ref, out_ref, acc_ref):
    j = pl.program_id(0)

    @pl.when(j == 0)
    def _init():
        acc_ref[0] = 0.0
        acc_ref[1] = 0.0
        acc_ref[2] = 0.0
        acc_ref[3] = 0.0

    x = cats_ref[...]
    gt = gt_ref[...]
    sel = sel_ref[...]
    posf = jnp.where(gt != _BG, 1.0, 0.0)

    sp = jnp.maximum(x, 0.0) + jnp.log1p(jnp.exp(-jnp.abs(x)))
    col = lax.broadcasted_iota(jnp.int32, x.shape, 1)
    rowsum = jnp.sum(jnp.where(col < _BG, sp, 0.0), axis=1, keepdims=True)
    xc = jnp.sum(jnp.where(col == gt, x, 0.0), axis=1, keepdims=True)

    pos_part = jnp.sum(posf * (rowsum - xc))
    neg_part = jnp.sum(sel * rowsum)

    d = bbs_ref[...] - gtb_ref[...]
    ad = jnp.abs(d)
    l1 = jnp.where(ad < 1.0, 0.5 * d * d, ad - 0.5)
    loc_part = jnp.sum(l1 * posf)
    np_part = jnp.sum(posf)

    acc_ref[0] += np_part
    acc_ref[1] += pos_part
    acc_ref[2] += neg_part
    acc_ref[3] += loc_part

    @pl.when(j == pl.num_programs(0) - 1)
    def _fini():
        n = acc_ref[0]
        conf = acc_ref[1] + acc_ref[2]
        loc = acc_ref[3]
        out_ref[0] = (conf + loc) / n
        out_ref[1] = loc
        out_ref[2] = conf


def _tc_loss(cats, bbs, gtb, gt2, sel2):
    return pl.pallas_call(
        _tc_body,
        grid=(_N // _R,),
        in_specs=[
            pl.BlockSpec((_R, _NUM_CLASSES), lambda j: (j, 0)),
            pl.BlockSpec((_R, 4), lambda j: (j, 0)),
            pl.BlockSpec((_R, 4), lambda j: (j, 0)),
            pl.BlockSpec((_R, 1), lambda j: (j, 0)),
            pl.BlockSpec((_R, 1), lambda j: (j, 0)),
        ],
        out_specs=pl.BlockSpec(memory_space=pltpu.SMEM),
        out_shape=jax.ShapeDtypeStruct((3,), jnp.float32),
        scratch_shapes=[pltpu.SMEM((4,), jnp.float32)],
    )(cats, bbs, gtb, gt2, sel2)


def _sel_scaffold(gt):
    neg = gt == _BG
    num_pos = jnp.sum((~neg).astype(jnp.int32))
    k = _RATIO * num_pos
    rank = jnp.cumsum(neg.astype(jnp.int32)) - 1
    return jnp.where(jnp.logical_and(neg, rank < k), 1.0, 0.0).astype(jnp.float32)


def kernel(bbs_preds, cats_preds, gt_bbs, gt_cats):
    gt = gt_cats.astype(jnp.int32)
    sel = _sel_scaffold(gt)
    out = _tc_loss(
        cats_preds,
        bbs_preds,
        gt_bbs,
        gt.reshape(_N, 1),
        sel.reshape(_N, 1),
    )
    return (out[0], out[1], out[2])


# speedup vs baseline: 11.0506x; 11.0506x over previous
import functools

import jax
import jax.numpy as jnp
from jax import lax
from jax.experimental import pallas as pl
from jax.experimental.pallas import tpu as pltpu

_NUM_CLASSES = 21
_BG = 20
_RATIO = 3
_N = 131072
_C = 8192


def _tc_body(cats_ref, bbs_ref, gtb_ref, gt_ref, sel_ref, out_ref, acc_ref):
    j = pl.program_id(0)

    @pl.when(j == 0)
    def _init():
        acc_ref[0] = 0.0
        acc_ref[1] = 0.0
        acc_ref[2] = 0.0
        acc_ref[3] = 0.0

    x = cats_ref[...]
    gt = gt_ref[...]
    sel = sel_ref[...]
    posf = jnp.where(gt != _BG, 1.0, 0.0)

    sp = jnp.maximum(x, 0.0) + jnp.log1p(jnp.exp(-jnp.abs(x)))
    row = lax.broadcasted_iota(jnp.int32, x.shape, 0)
    w = posf + sel
    conf_part = jnp.sum(jnp.where(row < _BG, sp, 0.0) * w)
    xc_part = jnp.sum(jnp.where(row == gt, x, 0.0) * posf)

    d = bbs_ref[...] - gtb_ref[...]
    ad = jnp.abs(d)
    l1 = jnp.where(ad < 1.0, 0.5 * d * d, ad - 0.5)
    loc_part = jnp.sum(l1 * posf)
    np_part = jnp.sum(posf)

    acc_ref[0] += np_part
    acc_ref[1] += conf_part - xc_part
    acc_ref[2] += loc_part
    acc_ref[3] += 0.0

    @pl.when(j == pl.num_programs(0) - 1)
    def _fini():
        n = acc_ref[0]
        conf = acc_ref[1]
        loc = acc_ref[2]
        out_ref[0] = (conf + loc) / n
        out_ref[1] = loc
        out_ref[2] = conf


def _tc_loss(catsT, bbsT, gtbT, gt1, sel1):
    return pl.pallas_call(
        _tc_body,
        grid=(_N // _C,),
        in_specs=[
            pl.BlockSpec((_NUM_CLASSES, _C), lambda j: (0, j)),
            pl.BlockSpec((4, _C), lambda j: (0, j)),
            pl.BlockSpec((4, _C), lambda j: (0, j)),
            pl.BlockSpec((1, _C), lambda j: (0, j)),
            pl.BlockSpec((1, _C), lambda j: (0, j)),
        ],
        out_specs=pl.BlockSpec(memory_space=pltpu.SMEM),
        out_shape=jax.ShapeDtypeStruct((3,), jnp.float32),
        scratch_shapes=[pltpu.SMEM((4,), jnp.float32)],
    )(catsT, bbsT, gtbT, gt1, sel1)


def _sel_scaffold(gt):
    neg = gt == _BG
    num_pos = jnp.sum((~neg).astype(jnp.int32))
    k = _RATIO * num_pos
    rank = jnp.cumsum(neg.astype(jnp.int32)) - 1
    return jnp.where(jnp.logical_and(neg, rank < k), 1.0, 0.0).astype(jnp.float32)


def kernel(bbs_preds, cats_preds, gt_bbs, gt_cats):
    gt = gt_cats.astype(jnp.int32)
    sel = _sel_scaffold(gt)
    out = _tc_loss(
        cats_preds.T,
        bbs_preds.T,
        gt_bbs.T,
        gt.reshape(1, _N),
        sel.reshape(1, _N),
    )
    return (out[0], out[1], out[2])
